# baseline (speedup 1.0000x reference)
import functools

import jax
import jax.numpy as jnp
from jax import lax
from jax.experimental import pallas as pl
from jax.experimental.pallas import tpu as pltpu
from jax.experimental.pallas import tpu_sc as plsc

N_NODES = 10000
N_EDGES = 160000
N_GRAPHS = 16

NN = 10240
EPAD = 163840
NW = 32
NS = 16
EP = EPAD // NW
CH = 128
NCH = EP // CH
NP = NN // NW
NEG = -3.0e38

_ROW_BLK = 640

_SC_PARAMS = pltpu.CompilerParams(needs_layout_passes=False)


def _qkvs_body(nx, act, d, nvb, *refs):
    x_refs = refs[:nx]
    w_ref, b_ref = refs[nx], refs[nx + 1]
    outs = refs[nx + 2:]
    x = x_refs[0][...] if nx == 1 else jnp.concatenate(
        [r[...] for r in x_refs], axis=1)
    if act:
        x = jnp.where(x > 0, x, jnp.exp(jnp.minimum(x, 0.0)) - 1.0)
    full = jnp.dot(x, w_ref[...], preferred_element_type=jnp.float32) + b_ref[...]
    outs[0][0] = full[:, :d]
    outs[0][1] = full[:, d:2 * d]
    outs[1][...] = full[:, 3 * d:]
    for i in range(nvb):
        outs[2 + i][...] = full[:, 2 * d + i * 128:2 * d + (i + 1) * 128]


def _qkvs_matmul(xs, w, b, act):
    nx = len(xs)
    din = sum(x.shape[1] for x in xs)
    d = w.shape[1] // 4
    nvb = d // 128
    grid = NN // _ROW_BLK
    in_specs = [pl.BlockSpec((_ROW_BLK, x.shape[1]), lambda i: (i, 0)) for x in xs]
    in_specs += [
        pl.BlockSpec((din, 4 * d), lambda i: (0, 0)),
        pl.BlockSpec((1, 4 * d), lambda i: (0, 0)),
    ]
    out_specs = [pl.BlockSpec((2, _ROW_BLK, d), lambda i: (0, i, 0)),
                 pl.BlockSpec((_ROW_BLK, d), lambda i: (i, 0))] + [
        pl.BlockSpec((_ROW_BLK, 128), lambda i: (i, 0))] * nvb
    out_shape = [jax.ShapeDtypeStruct((2, NN, d), jnp.float32),
                 jax.ShapeDtypeStruct((NN, d), jnp.float32)] + [
        jax.ShapeDtypeStruct((NN, 128), jnp.float32)] * nvb
    return pl.pallas_call(
        functools.partial(_qkvs_body, nx, act, d, nvb),
        grid=(grid,),
        in_specs=in_specs,
        out_specs=out_specs,
        out_shape=out_shape,
    )(*xs, w, b.reshape(1, -1))


def _combine_body(nvb, *refs):
    s_ref = refs[0]
    aggs = refs[1:1 + nvb]
    o_ref = refs[1 + nvb]
    out = s_ref[...]
    parts = []
    for i in range(nvb):
        parts.append(aggs[i][0] + aggs[i][1])
    o_ref[...] = out + jnp.concatenate(parts, axis=1)


def _combine(skip, agg_parts):
    d = skip.shape[1]
    nvb = d // 128
    grid = NN // _ROW_BLK
    in_specs = [pl.BlockSpec((_ROW_BLK, d), lambda i: (i, 0))] + [
        pl.BlockSpec((2, _ROW_BLK, 128), lambda i: (0, i, 0))] * nvb
    return pl.pallas_call(
        functools.partial(_combine_body, nvb),
        grid=(grid,),
        in_specs=in_specs,
        out_specs=pl.BlockSpec((_ROW_BLK, d), lambda i: (i, 0)),
        out_shape=jax.ShapeDtypeStruct((NN, d), jnp.float32),
    )(skip, *agg_parts)


def _gap_body(c2_ref, b_ref, o_ref):
    gids = lax.broadcasted_iota(jnp.int32, (N_GRAPHS, NN), 0)
    mask = (b_ref[...] == gids).astype(jnp.float32)
    sums = jnp.dot(mask, c2_ref[...], preferred_element_type=jnp.float32)
    counts = jnp.sum(mask, axis=1, keepdims=True)
    o_ref[...] = sums / jnp.maximum(counts, 1.0)


def _gap(c2_pad, batch_pad):
    return pl.pallas_call(
        _gap_body,
        in_specs=[
            pl.BlockSpec((NN, c2_pad.shape[1]), lambda: (0, 0)),
            pl.BlockSpec((1, NN), lambda: (0, 0)),
        ],
        out_specs=pl.BlockSpec((N_GRAPHS, c2_pad.shape[1]), lambda: (0, 0)),
        out_shape=jax.ShapeDtypeStruct((N_GRAPHS, c2_pad.shape[1]), jnp.float32),
    )(c2_pad, batch_pad.reshape(1, -1))


_GDN = lax.GatherDimensionNumbers(
    offset_dims=(), collapsed_slice_dims=(0,), start_index_map=(0,))


def _lane_shift(x, s):
    idx = (lax.iota(jnp.int32, 16) + s) & 15
    return lax.gather(x, idx[:, None], dimension_numbers=_GDN,
                      slice_sizes=(1,),
                      mode=lax.GatherScatterMode.PROMISE_IN_BOUNDS)


def _exp0(x):
    return jnp.exp(jnp.maximum(x, -87.0))


def _butterfly_softmax(key, m, s):
    for sh in (1, 2, 4, 8):
        k2 = _lane_shift(key, sh)
        m2 = jnp.where(key == k2, _lane_shift(m, sh), NEG)
        s2 = jnp.where(key == k2, _lane_shift(s, sh), 0.0)
        mm = jnp.maximum(m, m2)
        s = s * _exp0(m - mm) + s2 * _exp0(m2 - mm)
        m = mm
    return m, s


def _worker_id():
    return lax.axis_index("c") * NS + lax.axis_index("s")


CHA = 64
NCHA = EP // CHA


def _alpha_body(d, qk_hbm, comb_hbm, alpha_hbm, mpart_hbm, spart_hbm,
                idxv0, idxv1, qkbuf0, qkbuf1, abuf0, abuf1, mloc, sloc,
                semi0, semi1, semg0, semg1, sema0, sema1):
    w = _worker_id()
    scale = 1.0 / float(d) ** 0.5
    idxv = (idxv0, idxv1)
    qkbuf = (qkbuf0, qkbuf1)
    abuf = (abuf0, abuf1)
    semi = (semi0, semi1)
    semg = (semg0, semg1)
    sema = (sema0, sema1)

    def init(i, _):
        mloc[pl.ds(i * 16, 16)] = jnp.full((16,), NEG, jnp.float32)
        sloc[pl.ds(i * 16, 16)] = jnp.zeros((16,), jnp.float32)
        return 0
    lax.fori_loop(0, NN // 16, init, 0)

    def issue_idx(ci, b):
        base = (w * NCHA + ci) * 2 * CHA
        pltpu.async_copy(comb_hbm.at[pl.ds(base, 2 * CHA)], idxv[b], semi[b])

    def wait_idx(b):
        pltpu.make_async_copy(comb_hbm.at[pl.ds(0, 2 * CHA)], idxv[b],
                              semi[b]).wait()

    def issue_gather(b):
        pltpu.async_copy(qk_hbm.at[idxv[b]], qkbuf[b], semg[b])

    def wait_gather(b):
        pltpu.make_async_copy(qk_hbm.at[idxv[b]], qkbuf[b], semg[b]).wait()

    def wait_alpha(b):
        pltpu.make_async_copy(abuf[b], alpha_hbm.at[pl.ds(0, CHA)],
                              sema[b]).wait()

    def compute(ci, b):
        def grp(g, _):
            lanes = lax.iota(jnp.int32, 16) + g * 16

            def dot(jo, acc):
                for u in range(16):
                    jv = jnp.full((16,), jo * 16 + u, jnp.int32)
                    acc = acc + (plsc.load_gather(qkbuf[b], [lanes, jv]) *
                                 plsc.load_gather(qkbuf[b], [lanes + CHA, jv]))
                return acc
            alpha = lax.fori_loop(0, d // 16, dot,
                                  jnp.zeros((16,), jnp.float32)) * scale
            dst16 = idxv[b][pl.ds(g * 16, 16)]
            m, ss = _butterfly_softmax(dst16, alpha,
                                       jnp.ones((16,), jnp.float32))
            curm = plsc.load_gather(mloc, [dst16])
            curs = plsc.load_gather(sloc, [dst16])
            mm = jnp.maximum(curm, m)
            snew = curs * _exp0(curm - mm) + ss * _exp0(m - mm)
            plsc.store_scatter(mloc, [dst16], mm)
            plsc.store_scatter(sloc, [dst16], snew)
            abuf[b][pl.ds(g * 16, 16)] = alpha
            return 0
        lax.fori_loop(0, CHA // 16, grp, 0)
        pltpu.async_copy(abuf[b], alpha_hbm.at[pl.ds(w * EP + ci * CHA, CHA)],
                         sema[b])

    issue_idx(0, 0)
    issue_idx(1, 1)
    wait_idx(0)
    issue_gather(0)
    wait_idx(1)
    issue_gather(1)

    def pair(i, _):
        c0 = 2 * i
        wait_gather(0)

        @pl.when(i > 0)
        def _():
            wait_alpha(0)
        compute(c0, 0)
        issue_idx(c0 + 2, 0)
        wait_gather(1)

        @pl.when(i > 0)
        def _():
            wait_alpha(1)
        compute(c0 + 1, 1)
        issue_idx(c0 + 3, 1)
        wait_idx(0)
        issue_gather(0)
        wait_idx(1)
        issue_gather(1)
        return 0
    lax.fori_loop(0, NCHA // 2 - 1, pair, 0)
    wait_gather(0)
    wait_alpha(0)
    compute(NCHA - 2, 0)
    wait_gather(1)
    wait_alpha(1)
    compute(NCHA - 1, 1)
    wait_alpha(0)
    wait_alpha(1)
    pltpu.sync_copy(mloc, mpart_hbm.at[w])
    pltpu.sync_copy(sloc, spart_hbm.at[w])


def _alpha_kernel(d):
    mesh = plsc.VectorSubcoreMesh(core_axis_name="c", subcore_axis_name="s")
    return pl.kernel(
        functools.partial(_alpha_body, d),
        out_type=(
            jax.ShapeDtypeStruct((EPAD,), jnp.float32),
            jax.ShapeDtypeStruct((NW, NN), jnp.float32),
            jax.ShapeDtypeStruct((NW, NN), jnp.float32),
        ),
        mesh=mesh,
        scratch_types=[
            pltpu.VMEM((2 * CHA,), jnp.int32),
            pltpu.VMEM((2 * CHA,), jnp.int32),
            pltpu.VMEM((2 * CHA, d), jnp.float32),
            pltpu.VMEM((2 * CHA, d), jnp.float32),
            pltpu.VMEM((CHA,), jnp.float32),
            pltpu.VMEM((CHA,), jnp.float32),
            pltpu.VMEM((NN,), jnp.float32),
            pltpu.VMEM((NN,), jnp.float32),
            pltpu.SemaphoreType.DMA,
            pltpu.SemaphoreType.DMA,
            pltpu.SemaphoreType.DMA,
            pltpu.SemaphoreType.DMA,
            pltpu.SemaphoreType.DMA,
            pltpu.SemaphoreType.DMA,
        ],
        compiler_params=_SC_PARAMS,
    )


CHW = 1024
NCHW = EP // CHW


def _wgt_body(dst_hbm, alpha_hbm, mg_hbm, sg_hbm, wgt_hbm,
              dstb, ab, wb, mv, sv):
    w = _worker_id()
    pltpu.sync_copy(mg_hbm, mv)
    pltpu.sync_copy(sg_hbm, sv)

    def chunk(ci, _):
        base = w * EP + ci * CHW
        pltpu.sync_copy(dst_hbm.at[pl.ds(base, CHW)], dstb)
        pltpu.sync_copy(alpha_hbm.at[pl.ds(base, CHW)], ab)

        def grp(g, _):
            sl = pl.ds(g * 16, 16)
            dst16 = dstb[sl]
            m16 = plsc.load_gather(mv, [dst16])
            s16 = plsc.load_gather(sv, [dst16])
            wb[sl] = _exp0(ab[sl] - m16) / (s16 + 1e-16)
            return 0
        lax.fori_loop(0, CHW // 16, grp, 0)
        pltpu.sync_copy(wb, wgt_hbm.at[pl.ds(base, CHW)])
        return 0
    lax.fori_loop(0, NCHW, chunk, 0)


def _wgt_kernel():
    mesh = plsc.VectorSubcoreMesh(core_axis_name="c", subcore_axis_name="s")
    return pl.kernel(
        _wgt_body,
        out_type=jax.ShapeDtypeStruct((EPAD,), jnp.float32),
        mesh=mesh,
        scratch_types=[
            pltpu.VMEM((CHW,), jnp.int32),
            pltpu.VMEM((CHW,), jnp.float32),
            pltpu.VMEM((CHW,), jnp.float32),
            pltpu.VMEM((NN,), jnp.float32),
            pltpu.VMEM((NN,), jnp.float32),
        ],
        compiler_params=_SC_PARAMS,
    )


def _merge_body(mpart_hbm, spart_hbm, mg_hbm, sg_hbm, blkm, blks, mgv, sgv):
    w = _worker_id()
    for t in range(NW):
        pltpu.sync_copy(mpart_hbm.at[pl.ds(t * NN + w * NP, NP)],
                        blkm.at[pl.ds(t * NP, NP)])
        pltpu.sync_copy(spart_hbm.at[pl.ds(t * NN + w * NP, NP)],
                        blks.at[pl.ds(t * NP, NP)])

    def col(i, _):
        m = jnp.full((16,), NEG, jnp.float32)
        for t in range(NW):
            m = jnp.maximum(m, blkm[pl.ds(t * NP + i * 16, 16)])
        s = jnp.zeros((16,), jnp.float32)
        for t in range(NW):
            mt = blkm[pl.ds(t * NP + i * 16, 16)]
            s = s + blks[pl.ds(t * NP + i * 16, 16)] * _exp0(mt - m)
        mgv[pl.ds(i * 16, 16)] = m
        sgv[pl.ds(i * 16, 16)] = s
        return 0
    lax.fori_loop(0, NP // 16, col, 0)
    pltpu.sync_copy(mgv, mg_hbm.at[pl.ds(w * NP, NP)])
    pltpu.sync_copy(sgv, sg_hbm.at[pl.ds(w * NP, NP)])


def _merge_kernel():
    mesh = plsc.VectorSubcoreMesh(core_axis_name="c", subcore_axis_name="s")
    return pl.kernel(
        _merge_body,
        out_type=(
            jax.ShapeDtypeStruct((NN,), jnp.float32),
            jax.ShapeDtypeStruct((NN,), jnp.float32),
        ),
        mesh=mesh,
        scratch_types=[
            pltpu.VMEM((NW * NP,), jnp.float32),
            pltpu.VMEM((NW * NP,), jnp.float32),
            pltpu.VMEM((NP,), jnp.float32),
            pltpu.VMEM((NP,), jnp.float32),
        ],
        compiler_params=_SC_PARAMS,
    )


CHC = 64
NCHC = EP // CHC


def _agg_body(nvb, *refs):
    v_blocks = refs[:nvb]
    comb_hbm, wgt_hbm = refs[nvb:nvb + 2]
    agg_outs = refs[nvb + 2:nvb + 2 + nvb]
    (idxv0, idxv1, srcv0, srcv1, dsts0, dsts1, wb0, wb1,
     vbuf0, vbuf1, sbuf0, sbuf1, zbuf, aggsp,
     semi0, semi1, semg0, semg1, sems0, sems1) = refs[nvb + 2 + nvb:]
    cid = lax.axis_index("c")
    sid = lax.axis_index("s")
    w = cid * NS + sid
    idxv = (idxv0, idxv1)
    srcv = (srcv0, srcv1)
    dsts = (dsts0, dsts1)
    wb = (wb0, wb1)
    vbuf = (vbuf0, vbuf1)
    sbuf = (sbuf0, sbuf1)
    semi = (semi0, semi1)
    semg = (semg0, semg1)
    sems = (sems0, sems1)

    def zrow(r, _):
        for jj in range(8):
            zbuf[r, pl.ds(jj * 16, 16)] = jnp.zeros((16,), jnp.float32)
        return 0
    lax.fori_loop(0, 8, zrow, 0)

    def issue_idx(ci, b):
        pltpu.async_copy(comb_hbm.at[pl.ds((w * NCHC + ci) * 2 * CHC,
                                           2 * CHC)], idxv[b], semi[b])
        pltpu.async_copy(wgt_hbm.at[pl.ds(w * EP + ci * CHC, CHC)],
                         wb[b], semi[b])

    def wait_idx_derive_src(b):
        pltpu.make_async_copy(comb_hbm.at[pl.ds(0, 2 * CHC)], idxv[b],
                              semi[b]).wait()
        pltpu.make_async_copy(wgt_hbm.at[pl.ds(0, CHC)], wb[b],
                              semi[b]).wait()
        for h in range(CHC // 16):
            hs = pl.ds(h * 16, 16)
            srcv[b][hs] = idxv[b][pl.ds(CHC + h * 16, 16)] - NN

    def wait_scat(b):
        pltpu.make_async_copy(sbuf[b], aggsp.at[dsts[b]], sems[b]).wait()

    for blk in range(nvb):
        vb_hbm = v_blocks[blk]

        def issue_gather(b, _vb=vb_hbm):
            pltpu.async_copy(_vb.at[srcv[b]], vbuf[b], semg[b])

        def wait_gather(b, _vb=vb_hbm):
            pltpu.make_async_copy(_vb.at[srcv[b]], vbuf[b], semg[b]).wait()

        def process(ci, b):
            def grp(g, _):
                w16 = wb[b][pl.ds(g * 16, 16)]
                for u in range(16):
                    r = g * 16 + u
                    wv = jnp.full((16,), w16[u])
                    for jj in range(8):
                        cs = pl.ds(jj * 16, 16)
                        sbuf[b][r, cs] = vbuf[b][r, cs] * wv
                return 0
            lax.fori_loop(0, CHC // 16, grp, 0)
            for h in range(CHC // 16):
                hs = pl.ds(h * 16, 16)
                dsts[b][hs] = idxv[b][hs]
            pltpu.async_copy(sbuf[b], aggsp.at[dsts[b]], sems[b], add=True)

        def zsp(i, _):
            pltpu.sync_copy(zbuf, aggsp.at[pl.ds(sid * (NN // NS) + i * 8, 8)])
            return 0
        lax.fori_loop(0, NN // NS // 8, zsp, 0)
        plsc.subcore_barrier()

        issue_idx(0, 0)
        issue_idx(1, 1)
        wait_idx_derive_src(0)
        issue_gather(0)
        wait_idx_derive_src(1)
        issue_gather(1)

        def pair(i, _):
            c0 = 2 * i
            wait_gather(0)

            @pl.when(i > 0)
            def _():
                wait_scat(0)
            process(c0, 0)
            issue_idx(c0 + 2, 0)
            wait_gather(1)

            @pl.when(i > 0)
            def _():
                wait_scat(1)
            process(c0 + 1, 1)
            issue_idx(c0 + 3, 1)
            wait_idx_derive_src(0)
            issue_gather(0)
            wait_idx_derive_src(1)
            issue_gather(1)
            return 0
        lax.fori_loop(0, NCHC // 2 - 1, pair, 0)
        wait_gather(0)
        wait_scat(0)
        process(NCHC - 2, 0)
        wait_gather(1)
        wait_scat(1)
        process(NCHC - 1, 1)
        wait_scat(0)
        wait_scat(1)
        plsc.subcore_barrier()

        def dump(i, _):
            rowbase = sid * (NN // NS) + i * 128
            pltpu.sync_copy(aggsp.at[pl.ds(rowbase, 128)],
                            agg_outs[blk].at[cid, pl.ds(rowbase, 128)])
            return 0
        lax.fori_loop(0, NN // NS // 128, dump, 0)
        plsc.subcore_barrier()


def _agg_kernel(d):
    nvb = d // 128
    mesh = plsc.VectorSubcoreMesh(core_axis_name="c", subcore_axis_name="s")
    return pl.kernel(
        functools.partial(_agg_body, nvb),
        out_type=tuple(
            jax.ShapeDtypeStruct((2, NN, 128), jnp.float32)
            for _ in range(nvb)),
        mesh=mesh,
        scratch_types=[
            pltpu.VMEM((2 * CHC,), jnp.int32),
            pltpu.VMEM((2 * CHC,), jnp.int32),
            pltpu.VMEM((CHC,), jnp.int32),
            pltpu.VMEM((CHC,), jnp.int32),
            pltpu.VMEM((CHC,), jnp.int32),
            pltpu.VMEM((CHC,), jnp.int32),
            pltpu.VMEM((CHC,), jnp.float32),
            pltpu.VMEM((CHC,), jnp.float32),
            pltpu.VMEM((CHC, 128), jnp.float32),
            pltpu.VMEM((CHC, 128), jnp.float32),
            pltpu.VMEM((CHC, 128), jnp.float32),
            pltpu.VMEM((CHC, 128), jnp.float32),
            pltpu.VMEM((8, 128), jnp.float32),
            pltpu.VMEM_SHARED((NN, 128), jnp.float32),
            pltpu.SemaphoreType.DMA,
            pltpu.SemaphoreType.DMA,
            pltpu.SemaphoreType.DMA,
            pltpu.SemaphoreType.DMA,
            pltpu.SemaphoreType.DMA,
            pltpu.SemaphoreType.DMA,
        ],
        compiler_params=_SC_PARAMS,
    )


def _layer(xs, p, comb, dst_p, act):
    d = p["Wq"].shape[1]
    w = jnp.concatenate([p["Wq"], p["Wk"], p["Wv"], p["Ws"]], axis=1)
    b = jnp.concatenate([p["bq"], p["bk"], p["bv"], p["bs"]])
    outs = _qkvs_matmul(xs, w, b, act)
    qk, skip = outs[0], outs[1]
    v_blocks = outs[2:]
    alpha, mpart, spart = _alpha_kernel(d)(qk.reshape(2 * NN, d), comb)
    mg, sg = _merge_kernel()(mpart.reshape(-1), spart.reshape(-1))
    wgt = _wgt_kernel()(dst_p, alpha, mg, sg)
    agg_parts = _agg_kernel(d)(*v_blocks, comb, wgt)
    if not isinstance(agg_parts, (list, tuple)):
        agg_parts = (agg_parts,)
    return _combine(skip, agg_parts)


def kernel(features, img_feat, edge_index, batch_index, params):
    pad_n = NN - N_NODES
    feat_p = jnp.pad(features, ((0, pad_n), (0, 0)))
    img_p = jnp.pad(img_feat, ((0, pad_n), (0, 0)))
    src_p = jnp.pad(edge_index[0], (0, EPAD - N_EDGES))
    dst_p = jnp.pad(edge_index[1], (0, EPAD - N_EDGES),
                    constant_values=N_NODES)
    batch_p = jnp.pad(batch_index, (0, pad_n), constant_values=N_GRAPHS)
    comb = jnp.concatenate(
        [dst_p.reshape(-1, CHA), src_p.reshape(-1, CHA) + NN],
        axis=1).reshape(-1)

    h1 = _layer((feat_p,), params["conv1"], comb, dst_p, act=False)
    h2 = _layer((h1,), params["conv2"], comb, dst_p, act=True)
    h3p = _layer((h2,), params["conv3"], comb, dst_p, act=False)
    h4 = _layer((h3p,), params["conv4"], comb, dst_p, act=True)
    img1p = _layer((img_p,), params["imgconv1"], comb, dst_p, act=False)
    img2 = _layer((img1p,), params["imgconv2"], comb, dst_p, act=True)
    img3p = _layer((img2,), params["imgconv3"], comb, dst_p, act=False)
    img4 = _layer((img3p,), params["imgconv4"], comb, dst_p, act=True)
    combine_p = _layer((h2, img2), params["neck"], comb, dst_p, act=False)
    c2 = _layer((combine_p,), params["neck2"], comb, dst_p, act=True)
    c3p = _layer((c2,), params["c3"], comb, dst_p, act=False)
    c4 = _layer((c3p,), params["c4"], comb, dst_p, act=True)
    hidden = _gap(c2, batch_p)
    return (h2[:N_NODES], img2[:N_NODES], c2[:N_NODES], h4[:N_NODES],
            img4[:N_NODES], c4[:N_NODES], hidden)


# speedup vs baseline: 1.7759x; 1.7759x over previous
import functools

import jax
import jax.numpy as jnp
from jax import lax
from jax.experimental import pallas as pl
from jax.experimental.pallas import tpu as pltpu
from jax.experimental.pallas import tpu_sc as plsc

N_NODES = 10000
N_EDGES = 160000
N_GRAPHS = 16

NN = 10240
EPAD = 163840
NW = 32
NS = 16
EP = EPAD // NW
CH = 128
NCH = EP // CH
NP = NN // NW
NEG = -3.0e38

_ROW_BLK = 640

_SC_PARAMS = pltpu.CompilerParams(needs_layout_passes=False)


def _qkvs_body(nx, act, d, nvb, *refs):
    x_refs = refs[:nx]
    w_ref, b_ref = refs[nx], refs[nx + 1]
    outs = refs[nx + 2:]
    x = x_refs[0][...] if nx == 1 else jnp.concatenate(
        [r[...] for r in x_refs], axis=1)
    if act:
        x = jnp.where(x > 0, x, jnp.exp(jnp.minimum(x, 0.0)) - 1.0)
    full = jnp.dot(x, w_ref[...], preferred_element_type=jnp.float32) + b_ref[...]
    outs[0][0] = full[:, :d]
    outs[0][1] = full[:, d:2 * d]
    outs[1][...] = full[:, 3 * d:]
    for i in range(nvb):
        outs[2 + i][...] = full[:, 2 * d + i * 128:2 * d + (i + 1) * 128]


def _qkvs_matmul(xs, w, b, act):
    nx = len(xs)
    din = sum(x.shape[1] for x in xs)
    d = w.shape[1] // 4
    nvb = d // 128
    grid = NN // _ROW_BLK
    in_specs = [pl.BlockSpec((_ROW_BLK, x.shape[1]), lambda i: (i, 0)) for x in xs]
    in_specs += [
        pl.BlockSpec((din, 4 * d), lambda i: (0, 0)),
        pl.BlockSpec((1, 4 * d), lambda i: (0, 0)),
    ]
    out_specs = [pl.BlockSpec((2, _ROW_BLK, d), lambda i: (0, i, 0)),
                 pl.BlockSpec((_ROW_BLK, d), lambda i: (i, 0))] + [
        pl.BlockSpec((_ROW_BLK, 128), lambda i: (i, 0))] * nvb
    out_shape = [jax.ShapeDtypeStruct((2, NN, d), jnp.float32),
                 jax.ShapeDtypeStruct((NN, d), jnp.float32)] + [
        jax.ShapeDtypeStruct((NN, 128), jnp.float32)] * nvb
    return pl.pallas_call(
        functools.partial(_qkvs_body, nx, act, d, nvb),
        grid=(grid,),
        in_specs=in_specs,
        out_specs=out_specs,
        out_shape=out_shape,
    )(*xs, w, b.reshape(1, -1))


def _combine_body(nvb, *refs):
    s_ref = refs[0]
    aggs = refs[1:1 + nvb]
    o_ref = refs[1 + nvb]
    out = s_ref[...]
    parts = []
    for i in range(nvb):
        parts.append(aggs[i][0] + aggs[i][1])
    o_ref[...] = out + jnp.concatenate(parts, axis=1)


def _combine(skip, agg_parts):
    d = skip.shape[1]
    nvb = d // 128
    grid = NN // _ROW_BLK
    in_specs = [pl.BlockSpec((_ROW_BLK, d), lambda i: (i, 0))] + [
        pl.BlockSpec((2, _ROW_BLK, 128), lambda i: (0, i, 0))] * nvb
    return pl.pallas_call(
        functools.partial(_combine_body, nvb),
        grid=(grid,),
        in_specs=in_specs,
        out_specs=pl.BlockSpec((_ROW_BLK, d), lambda i: (i, 0)),
        out_shape=jax.ShapeDtypeStruct((NN, d), jnp.float32),
    )(skip, *agg_parts)


def _gap_body(c2_ref, b_ref, o_ref):
    gids = lax.broadcasted_iota(jnp.int32, (N_GRAPHS, NN), 0)
    mask = (b_ref[...] == gids).astype(jnp.float32)
    sums = jnp.dot(mask, c2_ref[...], preferred_element_type=jnp.float32)
    counts = jnp.sum(mask, axis=1, keepdims=True)
    o_ref[...] = sums / jnp.maximum(counts, 1.0)


def _gap(c2_pad, batch_pad):
    return pl.pallas_call(
        _gap_body,
        in_specs=[
            pl.BlockSpec((NN, c2_pad.shape[1]), lambda: (0, 0)),
            pl.BlockSpec((1, NN), lambda: (0, 0)),
        ],
        out_specs=pl.BlockSpec((N_GRAPHS, c2_pad.shape[1]), lambda: (0, 0)),
        out_shape=jax.ShapeDtypeStruct((N_GRAPHS, c2_pad.shape[1]), jnp.float32),
    )(c2_pad, batch_pad.reshape(1, -1))


_GDN = lax.GatherDimensionNumbers(
    offset_dims=(), collapsed_slice_dims=(0,), start_index_map=(0,))


def _lane_shift(x, s):
    idx = (lax.iota(jnp.int32, 16) + s) & 15
    return lax.gather(x, idx[:, None], dimension_numbers=_GDN,
                      slice_sizes=(1,),
                      mode=lax.GatherScatterMode.PROMISE_IN_BOUNDS)


def _exp0(x):
    return jnp.exp(jnp.maximum(x, -87.0))


def _butterfly_softmax(key, m, s):
    for sh in (1, 2, 4, 8):
        k2 = _lane_shift(key, sh)
        m2 = jnp.where(key == k2, _lane_shift(m, sh), NEG)
        s2 = jnp.where(key == k2, _lane_shift(s, sh), 0.0)
        mm = jnp.maximum(m, m2)
        s = s * _exp0(m - mm) + s2 * _exp0(m2 - mm)
        m = mm
    return m, s


def _worker_id():
    return lax.axis_index("c") * NS + lax.axis_index("s")


CHA = 64
NCHA = EP // CHA


def _alpha_body(d, qk_hbm, comb_hbm, alpha_hbm, mpart_hbm, spart_hbm,
                idxv0, idxv1, qkbuf0, qkbuf1, abuf0, abuf1, mloc, sloc,
                semi0, semi1, semg0, semg1, sema0, sema1):
    w = _worker_id()
    scale = 1.0 / float(d) ** 0.5
    idxv = (idxv0, idxv1)
    qkbuf = (qkbuf0, qkbuf1)
    abuf = (abuf0, abuf1)
    semi = (semi0, semi1)
    semg = (semg0, semg1)
    sema = (sema0, sema1)

    def init(i, _):
        mloc[pl.ds(i * 16, 16)] = jnp.full((16,), NEG, jnp.float32)
        sloc[pl.ds(i * 16, 16)] = jnp.zeros((16,), jnp.float32)
        return 0
    lax.fori_loop(0, NN // 16, init, 0)

    def issue_idx(ci, b):
        base = (w * NCHA + ci) * 2 * CHA
        pltpu.async_copy(comb_hbm.at[pl.ds(base, 2 * CHA)], idxv[b], semi[b])

    def wait_idx(b):
        pltpu.make_async_copy(comb_hbm.at[pl.ds(0, 2 * CHA)], idxv[b],
                              semi[b]).wait()

    def issue_gather(b):
        pltpu.async_copy(qk_hbm.at[idxv[b]], qkbuf[b], semg[b])

    def wait_gather(b):
        pltpu.make_async_copy(qk_hbm.at[idxv[b]], qkbuf[b], semg[b]).wait()

    def wait_alpha(b):
        pltpu.make_async_copy(abuf[b], alpha_hbm.at[pl.ds(0, CHA)],
                              sema[b]).wait()

    def compute(ci, b):
        def grp(g, _):
            iota = lax.iota(jnp.int32, 16)
            alpha = jnp.zeros((16,), jnp.float32)
            for u in range(16):
                r = g * 16 + u

                def rowdot(jo, acc, _r=r):
                    for jj in range(8):
                        sl = pl.ds(jo * 128 + jj * 16, 16)
                        acc = acc + qkbuf[b][_r, sl] * qkbuf[b][_r + CHA, sl]
                    return acc
                acc = lax.fori_loop(0, d // 128, rowdot,
                                    jnp.zeros((16,), jnp.float32))
                for sh in (1, 2, 4, 8):
                    acc = acc + _lane_shift(acc, sh)
                alpha = jnp.where(iota == u, acc, alpha)
            alpha = alpha * scale
            dst16 = idxv[b][pl.ds(g * 16, 16)]
            m, ss = _butterfly_softmax(dst16, alpha,
                                       jnp.ones((16,), jnp.float32))
            curm = plsc.load_gather(mloc, [dst16])
            curs = plsc.load_gather(sloc, [dst16])
            mm = jnp.maximum(curm, m)
            snew = curs * _exp0(curm - mm) + ss * _exp0(m - mm)
            plsc.store_scatter(mloc, [dst16], mm)
            plsc.store_scatter(sloc, [dst16], snew)
            abuf[b][pl.ds(g * 16, 16)] = alpha
            return 0
        lax.fori_loop(0, CHA // 16, grp, 0)
        pltpu.async_copy(abuf[b], alpha_hbm.at[pl.ds(w * EP + ci * CHA, CHA)],
                         sema[b])

    issue_idx(0, 0)
    issue_idx(1, 1)
    wait_idx(0)
    issue_gather(0)
    wait_idx(1)
    issue_gather(1)

    def pair(i, _):
        c0 = 2 * i
        wait_gather(0)

        @pl.when(i > 0)
        def _():
            wait_alpha(0)
        compute(c0, 0)
        issue_idx(c0 + 2, 0)
        wait_gather(1)

        @pl.when(i > 0)
        def _():
            wait_alpha(1)
        compute(c0 + 1, 1)
        issue_idx(c0 + 3, 1)
        wait_idx(0)
        issue_gather(0)
        wait_idx(1)
        issue_gather(1)
        return 0
    lax.fori_loop(0, NCHA // 2 - 1, pair, 0)
    wait_gather(0)
    wait_alpha(0)
    compute(NCHA - 2, 0)
    wait_gather(1)
    wait_alpha(1)
    compute(NCHA - 1, 1)
    wait_alpha(0)
    wait_alpha(1)
    pltpu.sync_copy(mloc, mpart_hbm.at[w])
    pltpu.sync_copy(sloc, spart_hbm.at[w])


def _alpha_kernel(d):
    mesh = plsc.VectorSubcoreMesh(core_axis_name="c", subcore_axis_name="s")
    return pl.kernel(
        functools.partial(_alpha_body, d),
        out_type=(
            jax.ShapeDtypeStruct((EPAD,), jnp.float32),
            jax.ShapeDtypeStruct((NW, NN), jnp.float32),
            jax.ShapeDtypeStruct((NW, NN), jnp.float32),
        ),
        mesh=mesh,
        scratch_types=[
            pltpu.VMEM((2 * CHA,), jnp.int32),
            pltpu.VMEM((2 * CHA,), jnp.int32),
            pltpu.VMEM((2 * CHA, d), jnp.float32),
            pltpu.VMEM((2 * CHA, d), jnp.float32),
            pltpu.VMEM((CHA,), jnp.float32),
            pltpu.VMEM((CHA,), jnp.float32),
            pltpu.VMEM((NN,), jnp.float32),
            pltpu.VMEM((NN,), jnp.float32),
            pltpu.SemaphoreType.DMA,
            pltpu.SemaphoreType.DMA,
            pltpu.SemaphoreType.DMA,
            pltpu.SemaphoreType.DMA,
            pltpu.SemaphoreType.DMA,
            pltpu.SemaphoreType.DMA,
        ],
        compiler_params=_SC_PARAMS,
    )


CHW = 1024
NCHW = EP // CHW


def _wgt_body(dst_hbm, alpha_hbm, mg_hbm, sg_hbm, wgt_hbm,
              dstb, ab, wb, mv, sv):
    w = _worker_id()
    pltpu.sync_copy(mg_hbm, mv)
    pltpu.sync_copy(sg_hbm, sv)

    def chunk(ci, _):
        base = w * EP + ci * CHW
        pltpu.sync_copy(dst_hbm.at[pl.ds(base, CHW)], dstb)
        pltpu.sync_copy(alpha_hbm.at[pl.ds(base, CHW)], ab)

        def grp(g, _):
            sl = pl.ds(g * 16, 16)
            dst16 = dstb[sl]
            m16 = plsc.load_gather(mv, [dst16])
            s16 = plsc.load_gather(sv, [dst16])
            wb[sl] = _exp0(ab[sl] - m16) / (s16 + 1e-16)
            return 0
        lax.fori_loop(0, CHW // 16, grp, 0)
        pltpu.sync_copy(wb, wgt_hbm.at[pl.ds(base, CHW)])
        return 0
    lax.fori_loop(0, NCHW, chunk, 0)


def _wgt_kernel():
    mesh = plsc.VectorSubcoreMesh(core_axis_name="c", subcore_axis_name="s")
    return pl.kernel(
        _wgt_body,
        out_type=jax.ShapeDtypeStruct((EPAD,), jnp.float32),
        mesh=mesh,
        scratch_types=[
            pltpu.VMEM((CHW,), jnp.int32),
            pltpu.VMEM((CHW,), jnp.float32),
            pltpu.VMEM((CHW,), jnp.float32),
            pltpu.VMEM((NN,), jnp.float32),
            pltpu.VMEM((NN,), jnp.float32),
        ],
        compiler_params=_SC_PARAMS,
    )


def _merge_body(mpart_hbm, spart_hbm, mg_hbm, sg_hbm, blkm, blks, mgv, sgv):
    w = _worker_id()
    for t in range(NW):
        pltpu.sync_copy(mpart_hbm.at[pl.ds(t * NN + w * NP, NP)],
                        blkm.at[pl.ds(t * NP, NP)])
        pltpu.sync_copy(spart_hbm.at[pl.ds(t * NN + w * NP, NP)],
                        blks.at[pl.ds(t * NP, NP)])

    def col(i, _):
        m = jnp.full((16,), NEG, jnp.float32)
        for t in range(NW):
            m = jnp.maximum(m, blkm[pl.ds(t * NP + i * 16, 16)])
        s = jnp.zeros((16,), jnp.float32)
        for t in range(NW):
            mt = blkm[pl.ds(t * NP + i * 16, 16)]
            s = s + blks[pl.ds(t * NP + i * 16, 16)] * _exp0(mt - m)
        mgv[pl.ds(i * 16, 16)] = m
        sgv[pl.ds(i * 16, 16)] = s
        return 0
    lax.fori_loop(0, NP // 16, col, 0)
    pltpu.sync_copy(mgv, mg_hbm.at[pl.ds(w * NP, NP)])
    pltpu.sync_copy(sgv, sg_hbm.at[pl.ds(w * NP, NP)])


def _merge_kernel():
    mesh = plsc.VectorSubcoreMesh(core_axis_name="c", subcore_axis_name="s")
    return pl.kernel(
        _merge_body,
        out_type=(
            jax.ShapeDtypeStruct((NN,), jnp.float32),
            jax.ShapeDtypeStruct((NN,), jnp.float32),
        ),
        mesh=mesh,
        scratch_types=[
            pltpu.VMEM((NW * NP,), jnp.float32),
            pltpu.VMEM((NW * NP,), jnp.float32),
            pltpu.VMEM((NP,), jnp.float32),
            pltpu.VMEM((NP,), jnp.float32),
        ],
        compiler_params=_SC_PARAMS,
    )


CHC = 64
NCHC = EP // CHC


def _agg_body(nvb, *refs):
    v_blocks = refs[:nvb]
    comb_hbm, wgt_hbm = refs[nvb:nvb + 2]
    agg_outs = refs[nvb + 2:nvb + 2 + nvb]
    (idxv0, idxv1, srcv0, srcv1, dsts0, dsts1, wb0, wb1,
     vbuf0, vbuf1, sbuf0, sbuf1, zbuf, aggsp,
     semi0, semi1, semg0, semg1, sems0, sems1) = refs[nvb + 2 + nvb:]
    cid = lax.axis_index("c")
    sid = lax.axis_index("s")
    w = cid * NS + sid
    idxv = (idxv0, idxv1)
    srcv = (srcv0, srcv1)
    dsts = (dsts0, dsts1)
    wb = (wb0, wb1)
    vbuf = (vbuf0, vbuf1)
    sbuf = (sbuf0, sbuf1)
    semi = (semi0, semi1)
    semg = (semg0, semg1)
    sems = (sems0, sems1)

    def zrow(r, _):
        for jj in range(8):
            zbuf[r, pl.ds(jj * 16, 16)] = jnp.zeros((16,), jnp.float32)
        return 0
    lax.fori_loop(0, 8, zrow, 0)

    def issue_idx(ci, b):
        pltpu.async_copy(comb_hbm.at[pl.ds((w * NCHC + ci) * 2 * CHC,
                                           2 * CHC)], idxv[b], semi[b])
        pltpu.async_copy(wgt_hbm.at[pl.ds(w * EP + ci * CHC, CHC)],
                         wb[b], semi[b])

    def wait_idx_derive_src(b):
        pltpu.make_async_copy(comb_hbm.at[pl.ds(0, 2 * CHC)], idxv[b],
                              semi[b]).wait()
        pltpu.make_async_copy(wgt_hbm.at[pl.ds(0, CHC)], wb[b],
                              semi[b]).wait()
        for h in range(CHC // 16):
            hs = pl.ds(h * 16, 16)
            srcv[b][hs] = idxv[b][pl.ds(CHC + h * 16, 16)] - NN

    def wait_scat(b):
        pltpu.make_async_copy(sbuf[b], aggsp.at[dsts[b]], sems[b]).wait()

    for blk in range(nvb):
        vb_hbm = v_blocks[blk]

        def issue_gather(b, _vb=vb_hbm):
            pltpu.async_copy(_vb.at[srcv[b]], vbuf[b], semg[b])

        def wait_gather(b, _vb=vb_hbm):
            pltpu.make_async_copy(_vb.at[srcv[b]], vbuf[b], semg[b]).wait()

        def process(ci, b):
            def grp(g, _):
                w16 = wb[b][pl.ds(g * 16, 16)]
                for u in range(16):
                    r = g * 16 + u
                    wv = jnp.full((16,), w16[u])
                    for jj in range(8):
                        cs = pl.ds(jj * 16, 16)
                        sbuf[b][r, cs] = vbuf[b][r, cs] * wv
                return 0
            lax.fori_loop(0, CHC // 16, grp, 0)
            for h in range(CHC // 16):
                hs = pl.ds(h * 16, 16)
                dsts[b][hs] = idxv[b][hs]
            pltpu.async_copy(sbuf[b], aggsp.at[dsts[b]], sems[b], add=True)

        def zsp(i, _):
            pltpu.sync_copy(zbuf, aggsp.at[pl.ds(sid * (NN // NS) + i * 8, 8)])
            return 0
        lax.fori_loop(0, NN // NS // 8, zsp, 0)
        plsc.subcore_barrier()

        issue_idx(0, 0)
        issue_idx(1, 1)
        wait_idx_derive_src(0)
        issue_gather(0)
        wait_idx_derive_src(1)
        issue_gather(1)

        def pair(i, _):
            c0 = 2 * i
            wait_gather(0)

            @pl.when(i > 0)
            def _():
                wait_scat(0)
            process(c0, 0)
            issue_idx(c0 + 2, 0)
            wait_gather(1)

            @pl.when(i > 0)
            def _():
                wait_scat(1)
            process(c0 + 1, 1)
            issue_idx(c0 + 3, 1)
            wait_idx_derive_src(0)
            issue_gather(0)
            wait_idx_derive_src(1)
            issue_gather(1)
            return 0
        lax.fori_loop(0, NCHC // 2 - 1, pair, 0)
        wait_gather(0)
        wait_scat(0)
        process(NCHC - 2, 0)
        wait_gather(1)
        wait_scat(1)
        process(NCHC - 1, 1)
        wait_scat(0)
        wait_scat(1)
        plsc.subcore_barrier()

        def dump(i, _):
            rowbase = sid * (NN // NS) + i * 128
            pltpu.sync_copy(aggsp.at[pl.ds(rowbase, 128)],
                            agg_outs[blk].at[cid, pl.ds(rowbase, 128)])
            return 0
        lax.fori_loop(0, NN // NS // 128, dump, 0)
        plsc.subcore_barrier()


def _agg_kernel(d):
    nvb = d // 128
    mesh = plsc.VectorSubcoreMesh(core_axis_name="c", subcore_axis_name="s")
    return pl.kernel(
        functools.partial(_agg_body, nvb),
        out_type=tuple(
            jax.ShapeDtypeStruct((2, NN, 128), jnp.float32)
            for _ in range(nvb)),
        mesh=mesh,
        scratch_types=[
            pltpu.VMEM((2 * CHC,), jnp.int32),
            pltpu.VMEM((2 * CHC,), jnp.int32),
            pltpu.VMEM((CHC,), jnp.int32),
            pltpu.VMEM((CHC,), jnp.int32),
            pltpu.VMEM((CHC,), jnp.int32),
            pltpu.VMEM((CHC,), jnp.int32),
            pltpu.VMEM((CHC,), jnp.float32),
            pltpu.VMEM((CHC,), jnp.float32),
            pltpu.VMEM((CHC, 128), jnp.float32),
            pltpu.VMEM((CHC, 128), jnp.float32),
            pltpu.VMEM((CHC, 128), jnp.float32),
            pltpu.VMEM((CHC, 128), jnp.float32),
            pltpu.VMEM((8, 128), jnp.float32),
            pltpu.VMEM_SHARED((NN, 128), jnp.float32),
            pltpu.SemaphoreType.DMA,
            pltpu.SemaphoreType.DMA,
            pltpu.SemaphoreType.DMA,
            pltpu.SemaphoreType.DMA,
            pltpu.SemaphoreType.DMA,
            pltpu.SemaphoreType.DMA,
        ],
        compiler_params=_SC_PARAMS,
    )


def _layer(xs, p, comb, dst_p, act):
    d = p["Wq"].shape[1]
    w = jnp.concatenate([p["Wq"], p["Wk"], p["Wv"], p["Ws"]], axis=1)
    b = jnp.concatenate([p["bq"], p["bk"], p["bv"], p["bs"]])
    outs = _qkvs_matmul(xs, w, b, act)
    qk, skip = outs[0], outs[1]
    v_blocks = outs[2:]
    alpha, mpart, spart = _alpha_kernel(d)(qk.reshape(2 * NN, d), comb)
    mg, sg = _merge_kernel()(mpart.reshape(-1), spart.reshape(-1))
    wgt = _wgt_kernel()(dst_p, alpha, mg, sg)
    agg_parts = _agg_kernel(d)(*v_blocks, comb, wgt)
    if not isinstance(agg_parts, (list, tuple)):
        agg_parts = (agg_parts,)
    return _combine(skip, agg_parts)


def kernel(features, img_feat, edge_index, batch_index, params):
    pad_n = NN - N_NODES
    feat_p = jnp.pad(features, ((0, pad_n), (0, 0)))
    img_p = jnp.pad(img_feat, ((0, pad_n), (0, 0)))
    src_p = jnp.pad(edge_index[0], (0, EPAD - N_EDGES))
    dst_p = jnp.pad(edge_index[1], (0, EPAD - N_EDGES),
                    constant_values=N_NODES)
    batch_p = jnp.pad(batch_index, (0, pad_n), constant_values=N_GRAPHS)
    comb = jnp.concatenate(
        [dst_p.reshape(-1, CHA), src_p.reshape(-1, CHA) + NN],
        axis=1).reshape(-1)

    h1 = _layer((feat_p,), params["conv1"], comb, dst_p, act=False)
    h2 = _layer((h1,), params["conv2"], comb, dst_p, act=True)
    h3p = _layer((h2,), params["conv3"], comb, dst_p, act=False)
    h4 = _layer((h3p,), params["conv4"], comb, dst_p, act=True)
    img1p = _layer((img_p,), params["imgconv1"], comb, dst_p, act=False)
    img2 = _layer((img1p,), params["imgconv2"], comb, dst_p, act=True)
    img3p = _layer((img2,), params["imgconv3"], comb, dst_p, act=False)
    img4 = _layer((img3p,), params["imgconv4"], comb, dst_p, act=True)
    combine_p = _layer((h2, img2), params["neck"], comb, dst_p, act=False)
    c2 = _layer((combine_p,), params["neck2"], comb, dst_p, act=True)
    c3p = _layer((c2,), params["c3"], comb, dst_p, act=False)
    c4 = _layer((c3p,), params["c4"], comb, dst_p, act=True)
    hidden = _gap(c2, batch_p)
    return (h2[:N_NODES], img2[:N_NODES], c2[:N_NODES], h4[:N_NODES],
            img4[:N_NODES], c4[:N_NODES], hidden)


# speedup vs baseline: 1.9359x; 1.0901x over previous
import functools

import jax
import jax.numpy as jnp
from jax import lax
from jax.experimental import pallas as pl
from jax.experimental.pallas import tpu as pltpu
from jax.experimental.pallas import tpu_sc as plsc

N_NODES = 10000
N_EDGES = 160000
N_GRAPHS = 16

NN = 10240
EPAD = 163840
NW = 32
NS = 16
EP = EPAD // NW
CH = 128
NCH = EP // CH
NP = NN // NW
NEG = -3.0e38

_ROW_BLK = 640

_SC_PARAMS = pltpu.CompilerParams(needs_layout_passes=False)


def _qkvs_body(nx, act, d, nvb, *refs):
    x_refs = refs[:nx]
    w_ref, b_ref = refs[nx], refs[nx + 1]
    outs = refs[nx + 2:]
    x = x_refs[0][...] if nx == 1 else jnp.concatenate(
        [r[...] for r in x_refs], axis=1)
    if act:
        x = jnp.where(x > 0, x, jnp.exp(jnp.minimum(x, 0.0)) - 1.0)
    full = jnp.dot(x, w_ref[...], preferred_element_type=jnp.float32) + b_ref[...]
    outs[0][0] = full[:, :d]
    outs[0][1] = full[:, d:2 * d]
    outs[1][...] = full[:, 3 * d:]
    for i in range(nvb):
        outs[2 + i][...] = full[:, 2 * d + i * 128:2 * d + (i + 1) * 128]


def _qkvs_matmul(xs, w, b, act):
    nx = len(xs)
    din = sum(x.shape[1] for x in xs)
    d = w.shape[1] // 4
    nvb = d // 128
    grid = NN // _ROW_BLK
    in_specs = [pl.BlockSpec((_ROW_BLK, x.shape[1]), lambda i: (i, 0)) for x in xs]
    in_specs += [
        pl.BlockSpec((din, 4 * d), lambda i: (0, 0)),
        pl.BlockSpec((1, 4 * d), lambda i: (0, 0)),
    ]
    out_specs = [pl.BlockSpec((2, _ROW_BLK, d), lambda i: (0, i, 0)),
                 pl.BlockSpec((_ROW_BLK, d), lambda i: (i, 0))] + [
        pl.BlockSpec((_ROW_BLK, 128), lambda i: (i, 0))] * nvb
    out_shape = [jax.ShapeDtypeStruct((2, NN, d), jnp.float32),
                 jax.ShapeDtypeStruct((NN, d), jnp.float32)] + [
        jax.ShapeDtypeStruct((NN, 128), jnp.float32)] * nvb
    return pl.pallas_call(
        functools.partial(_qkvs_body, nx, act, d, nvb),
        grid=(grid,),
        in_specs=in_specs,
        out_specs=out_specs,
        out_shape=out_shape,
    )(*xs, w, b.reshape(1, -1))


def _combine_body(nvb, *refs):
    s_ref = refs[0]
    aggs = refs[1:1 + nvb]
    o_ref = refs[1 + nvb]
    out = s_ref[...]
    parts = []
    for i in range(nvb):
        parts.append(aggs[i][0] + aggs[i][1])
    o_ref[...] = out + jnp.concatenate(parts, axis=1)


def _combine(skip, agg_parts):
    d = skip.shape[1]
    nvb = d // 128
    grid = NN // _ROW_BLK
    in_specs = [pl.BlockSpec((_ROW_BLK, d), lambda i: (i, 0))] + [
        pl.BlockSpec((2, _ROW_BLK, 128), lambda i: (0, i, 0))] * nvb
    return pl.pallas_call(
        functools.partial(_combine_body, nvb),
        grid=(grid,),
        in_specs=in_specs,
        out_specs=pl.BlockSpec((_ROW_BLK, d), lambda i: (i, 0)),
        out_shape=jax.ShapeDtypeStruct((NN, d), jnp.float32),
    )(skip, *agg_parts)


def _gap_body(c2_ref, b_ref, o_ref):
    gids = lax.broadcasted_iota(jnp.int32, (N_GRAPHS, NN), 0)
    mask = (b_ref[...] == gids).astype(jnp.float32)
    sums = jnp.dot(mask, c2_ref[...], preferred_element_type=jnp.float32)
    counts = jnp.sum(mask, axis=1, keepdims=True)
    o_ref[...] = sums / jnp.maximum(counts, 1.0)


def _gap(c2_pad, batch_pad):
    return pl.pallas_call(
        _gap_body,
        in_specs=[
            pl.BlockSpec((NN, c2_pad.shape[1]), lambda: (0, 0)),
            pl.BlockSpec((1, NN), lambda: (0, 0)),
        ],
        out_specs=pl.BlockSpec((N_GRAPHS, c2_pad.shape[1]), lambda: (0, 0)),
        out_shape=jax.ShapeDtypeStruct((N_GRAPHS, c2_pad.shape[1]), jnp.float32),
    )(c2_pad, batch_pad.reshape(1, -1))


_GDN = lax.GatherDimensionNumbers(
    offset_dims=(), collapsed_slice_dims=(0,), start_index_map=(0,))


def _lane_shift(x, s):
    idx = (lax.iota(jnp.int32, 16) + s) & 15
    return lax.gather(x, idx[:, None], dimension_numbers=_GDN,
                      slice_sizes=(1,),
                      mode=lax.GatherScatterMode.PROMISE_IN_BOUNDS)


def _exp0(x):
    return jnp.exp(jnp.maximum(x, -87.0))


def _butterfly_softmax(key, m, s):
    for sh in (1, 2, 4, 8):
        k2 = _lane_shift(key, sh)
        m2 = jnp.where(key == k2, _lane_shift(m, sh), NEG)
        s2 = jnp.where(key == k2, _lane_shift(s, sh), 0.0)
        mm = jnp.maximum(m, m2)
        s = s * _exp0(m - mm) + s2 * _exp0(m2 - mm)
        m = mm
    return m, s


def _worker_id():
    return lax.axis_index("c") * NS + lax.axis_index("s")


CHA = 32
NCHA = EP // CHA
NBA = 4


def _alpha_body(d, qk_hbm, comb_hbm, alpha_hbm, mpart_hbm, spart_hbm,
                idxv0, idxv1, idxv2, idxv3, dstc0, dstc1, dstc2, dstc3,
                qkbuf0, qkbuf1, qkbuf2, qkbuf3,
                abuf0, abuf1, abuf2, abuf3, mloc, sloc,
                semi0, semi1, semi2, semi3,
                semg0, semg1, semg2, semg3,
                sema0, sema1, sema2, sema3):
    w = _worker_id()
    scale = 1.0 / float(d) ** 0.5
    idxv = (idxv0, idxv1, idxv2, idxv3)
    dstc = (dstc0, dstc1, dstc2, dstc3)
    qkbuf = (qkbuf0, qkbuf1, qkbuf2, qkbuf3)
    abuf = (abuf0, abuf1, abuf2, abuf3)
    semi = (semi0, semi1, semi2, semi3)
    semg = (semg0, semg1, semg2, semg3)
    sema = (sema0, sema1, sema2, sema3)

    def init(i, _):
        mloc[pl.ds(i * 16, 16)] = jnp.full((16,), NEG, jnp.float32)
        sloc[pl.ds(i * 16, 16)] = jnp.zeros((16,), jnp.float32)
        return 0
    lax.fori_loop(0, NN // 16, init, 0)

    def issue_idx(ci, b):
        base = (w * NCHA + ci) * 2 * CHA
        pltpu.async_copy(comb_hbm.at[pl.ds(base, 2 * CHA)], idxv[b], semi[b])

    def wait_idx(b):
        pltpu.make_async_copy(comb_hbm.at[pl.ds(0, 2 * CHA)], idxv[b],
                              semi[b]).wait()

    def issue_gather(b):
        pltpu.async_copy(qk_hbm.at[idxv[b]], qkbuf[b], semg[b])

    def wait_gather(b):
        pltpu.make_async_copy(qk_hbm.at[idxv[b]], qkbuf[b], semg[b]).wait()

    def wait_alpha(b):
        pltpu.make_async_copy(abuf[b], alpha_hbm.at[pl.ds(0, CHA)],
                              sema[b]).wait()

    def compute(ci, b):
        def grp(g, _):
            iota = lax.iota(jnp.int32, 16)
            alpha = jnp.zeros((16,), jnp.float32)
            for u in range(16):
                r = g * 16 + u

                def rowdot(jo, acc, _r=r):
                    for jj in range(8):
                        sl = pl.ds(jo * 128 + jj * 16, 16)
                        acc = acc + qkbuf[b][_r, sl] * qkbuf[b][_r + CHA, sl]
                    return acc
                acc = lax.fori_loop(0, d // 128, rowdot,
                                    jnp.zeros((16,), jnp.float32))
                for sh in (1, 2, 4, 8):
                    acc = acc + _lane_shift(acc, sh)
                alpha = jnp.where(iota == u, acc, alpha)
            alpha = alpha * scale
            dst16 = dstc[b][pl.ds(g * 16, 16)]
            m, ss = _butterfly_softmax(dst16, alpha,
                                       jnp.ones((16,), jnp.float32))
            curm = plsc.load_gather(mloc, [dst16])
            curs = plsc.load_gather(sloc, [dst16])
            mm = jnp.maximum(curm, m)
            snew = curs * _exp0(curm - mm) + ss * _exp0(m - mm)
            plsc.store_scatter(mloc, [dst16], mm)
            plsc.store_scatter(sloc, [dst16], snew)
            abuf[b][pl.ds(g * 16, 16)] = alpha
            return 0
        lax.fori_loop(0, CHA // 16, grp, 0)
        pltpu.async_copy(abuf[b], alpha_hbm.at[pl.ds(w * EP + ci * CHA, CHA)],
                         sema[b])

    for b in range(NBA):
        issue_idx(b, b)
    for b in range(NBA):
        wait_idx(b)
        issue_gather(b)
    nloops = NCHA // NBA

    def ring(i, _):
        c0 = NBA * i
        for b in range(NBA):
            c = c0 + b
            wait_gather(b)
            for h in range(CHA // 16):
                hs = pl.ds(h * 16, 16)
                dstc[b][hs] = idxv[b][hs]

            @pl.when(i < nloops - 1)
            def _(b=b, c=c):
                issue_idx(c + NBA, b)

            @pl.when(i > 0)
            def _(b=b):
                wait_alpha(b)
            compute(c, b)

            @pl.when(i < nloops - 1)
            def _(b=b):
                wait_idx(b)
                issue_gather(b)
        return 0
    lax.fori_loop(0, nloops, ring, 0)
    for b in range(NBA):
        wait_alpha(b)
    pltpu.sync_copy(mloc, mpart_hbm.at[w])
    pltpu.sync_copy(sloc, spart_hbm.at[w])


def _alpha_kernel(d):
    mesh = plsc.VectorSubcoreMesh(core_axis_name="c", subcore_axis_name="s")
    return pl.kernel(
        functools.partial(_alpha_body, d),
        out_type=(
            jax.ShapeDtypeStruct((EPAD,), jnp.float32),
            jax.ShapeDtypeStruct((NW, NN), jnp.float32),
            jax.ShapeDtypeStruct((NW, NN), jnp.float32),
        ),
        mesh=mesh,
        scratch_types=(
            [pltpu.VMEM((2 * CHA,), jnp.int32)] * 4
            + [pltpu.VMEM((CHA,), jnp.int32)] * 4
            + [pltpu.VMEM((2 * CHA, d), jnp.float32)] * 4
            + [pltpu.VMEM((CHA,), jnp.float32)] * 4
            + [pltpu.VMEM((NN,), jnp.float32)] * 2
            + [pltpu.SemaphoreType.DMA] * 12
        ),
        compiler_params=_SC_PARAMS,
    )


CHW = 1024
NCHW = EP // CHW


def _wgt_body(dst_hbm, alpha_hbm, mg_hbm, sg_hbm, wgt_hbm,
              dstb, ab, wb, mv, sv):
    w = _worker_id()
    pltpu.sync_copy(mg_hbm, mv)
    pltpu.sync_copy(sg_hbm, sv)

    def chunk(ci, _):
        base = w * EP + ci * CHW
        pltpu.sync_copy(dst_hbm.at[pl.ds(base, CHW)], dstb)
        pltpu.sync_copy(alpha_hbm.at[pl.ds(base, CHW)], ab)

        def grp(g, _):
            sl = pl.ds(g * 16, 16)
            dst16 = dstb[sl]
            m16 = plsc.load_gather(mv, [dst16])
            s16 = plsc.load_gather(sv, [dst16])
            wb[sl] = _exp0(ab[sl] - m16) / (s16 + 1e-16)
            return 0
        lax.fori_loop(0, CHW // 16, grp, 0)
        pltpu.sync_copy(wb, wgt_hbm.at[pl.ds(base, CHW)])
        return 0
    lax.fori_loop(0, NCHW, chunk, 0)


def _wgt_kernel():
    mesh = plsc.VectorSubcoreMesh(core_axis_name="c", subcore_axis_name="s")
    return pl.kernel(
        _wgt_body,
        out_type=jax.ShapeDtypeStruct((EPAD,), jnp.float32),
        mesh=mesh,
        scratch_types=[
            pltpu.VMEM((CHW,), jnp.int32),
            pltpu.VMEM((CHW,), jnp.float32),
            pltpu.VMEM((CHW,), jnp.float32),
            pltpu.VMEM((NN,), jnp.float32),
            pltpu.VMEM((NN,), jnp.float32),
        ],
        compiler_params=_SC_PARAMS,
    )


def _merge_body(mpart_hbm, spart_hbm, mg_hbm, sg_hbm, blkm, blks, mgv, sgv):
    w = _worker_id()
    for t in range(NW):
        pltpu.sync_copy(mpart_hbm.at[pl.ds(t * NN + w * NP, NP)],
                        blkm.at[pl.ds(t * NP, NP)])
        pltpu.sync_copy(spart_hbm.at[pl.ds(t * NN + w * NP, NP)],
                        blks.at[pl.ds(t * NP, NP)])

    def col(i, _):
        m = jnp.full((16,), NEG, jnp.float32)
        for t in range(NW):
            m = jnp.maximum(m, blkm[pl.ds(t * NP + i * 16, 16)])
        s = jnp.zeros((16,), jnp.float32)
        for t in range(NW):
            mt = blkm[pl.ds(t * NP + i * 16, 16)]
            s = s + blks[pl.ds(t * NP + i * 16, 16)] * _exp0(mt - m)
        mgv[pl.ds(i * 16, 16)] = m
        sgv[pl.ds(i * 16, 16)] = s
        return 0
    lax.fori_loop(0, NP // 16, col, 0)
    pltpu.sync_copy(mgv, mg_hbm.at[pl.ds(w * NP, NP)])
    pltpu.sync_copy(sgv, sg_hbm.at[pl.ds(w * NP, NP)])


def _merge_kernel():
    mesh = plsc.VectorSubcoreMesh(core_axis_name="c", subcore_axis_name="s")
    return pl.kernel(
        _merge_body,
        out_type=(
            jax.ShapeDtypeStruct((NN,), jnp.float32),
            jax.ShapeDtypeStruct((NN,), jnp.float32),
        ),
        mesh=mesh,
        scratch_types=[
            pltpu.VMEM((NW * NP,), jnp.float32),
            pltpu.VMEM((NW * NP,), jnp.float32),
            pltpu.VMEM((NP,), jnp.float32),
            pltpu.VMEM((NP,), jnp.float32),
        ],
        compiler_params=_SC_PARAMS,
    )


CHC = 64
NCHC = EP // CHC


def _agg_body(nvb, *refs):
    v_blocks = refs[:nvb]
    comb_hbm, wgt_hbm = refs[nvb:nvb + 2]
    agg_outs = refs[nvb + 2:nvb + 2 + nvb]
    (idxv0, idxv1, srcv0, srcv1, dsts0, dsts1, wb0, wb1,
     vbuf0, vbuf1, sbuf0, sbuf1, zbuf, aggsp,
     semi0, semi1, semg0, semg1, sems0, sems1) = refs[nvb + 2 + nvb:]
    cid = lax.axis_index("c")
    sid = lax.axis_index("s")
    w = cid * NS + sid
    idxv = (idxv0, idxv1)
    srcv = (srcv0, srcv1)
    dsts = (dsts0, dsts1)
    wb = (wb0, wb1)
    vbuf = (vbuf0, vbuf1)
    sbuf = (sbuf0, sbuf1)
    semi = (semi0, semi1)
    semg = (semg0, semg1)
    sems = (sems0, sems1)

    def zrow(r, _):
        for jj in range(8):
            zbuf[r, pl.ds(jj * 16, 16)] = jnp.zeros((16,), jnp.float32)
        return 0
    lax.fori_loop(0, 8, zrow, 0)

    def issue_idx(ci, b):
        pltpu.async_copy(comb_hbm.at[pl.ds((w * NCHC + ci) * 2 * CHC,
                                           2 * CHC)], idxv[b], semi[b])
        pltpu.async_copy(wgt_hbm.at[pl.ds(w * EP + ci * CHC, CHC)],
                         wb[b], semi[b])

    def wait_idx_derive_src(b):
        pltpu.make_async_copy(comb_hbm.at[pl.ds(0, 2 * CHC)], idxv[b],
                              semi[b]).wait()
        pltpu.make_async_copy(wgt_hbm.at[pl.ds(0, CHC)], wb[b],
                              semi[b]).wait()
        for h in range(CHC // 16):
            hs = pl.ds(h * 16, 16)
            srcv[b][hs] = idxv[b][pl.ds(CHC + h * 16, 16)] - NN

    def wait_scat(b):
        pltpu.make_async_copy(sbuf[b], aggsp.at[dsts[b]], sems[b]).wait()

    for blk in range(nvb):
        vb_hbm = v_blocks[blk]

        def issue_gather(b, _vb=vb_hbm):
            pltpu.async_copy(_vb.at[srcv[b]], vbuf[b], semg[b])

        def wait_gather(b, _vb=vb_hbm):
            pltpu.make_async_copy(_vb.at[srcv[b]], vbuf[b], semg[b]).wait()

        def process(ci, b):
            def grp(g, _):
                w16 = wb[b][pl.ds(g * 16, 16)]
                for u in range(16):
                    r = g * 16 + u
                    wv = jnp.full((16,), w16[u])
                    for jj in range(8):
                        cs = pl.ds(jj * 16, 16)
                        sbuf[b][r, cs] = vbuf[b][r, cs] * wv
                return 0
            lax.fori_loop(0, CHC // 16, grp, 0)
            for h in range(CHC // 16):
                hs = pl.ds(h * 16, 16)
                dsts[b][hs] = idxv[b][hs]
            pltpu.async_copy(sbuf[b], aggsp.at[dsts[b]], sems[b], add=True)

        def zsp(i, _):
            pltpu.sync_copy(zbuf, aggsp.at[pl.ds(sid * (NN // NS) + i * 8, 8)])
            return 0
        lax.fori_loop(0, NN // NS // 8, zsp, 0)
        plsc.subcore_barrier()

        issue_idx(0, 0)
        issue_idx(1, 1)
        wait_idx_derive_src(0)
        issue_gather(0)
        wait_idx_derive_src(1)
        issue_gather(1)

        def pair(i, _):
            c0 = 2 * i
            wait_gather(0)

            @pl.when(i > 0)
            def _():
                wait_scat(0)
            process(c0, 0)
            issue_idx(c0 + 2, 0)
            wait_gather(1)

            @pl.when(i > 0)
            def _():
                wait_scat(1)
            process(c0 + 1, 1)
            issue_idx(c0 + 3, 1)
            wait_idx_derive_src(0)
            issue_gather(0)
            wait_idx_derive_src(1)
            issue_gather(1)
            return 0
        lax.fori_loop(0, NCHC // 2 - 1, pair, 0)
        wait_gather(0)
        wait_scat(0)
        process(NCHC - 2, 0)
        wait_gather(1)
        wait_scat(1)
        process(NCHC - 1, 1)
        wait_scat(0)
        wait_scat(1)
        plsc.subcore_barrier()

        def dump(i, _):
            rowbase = sid * (NN // NS) + i * 128
            pltpu.sync_copy(aggsp.at[pl.ds(rowbase, 128)],
                            agg_outs[blk].at[cid, pl.ds(rowbase, 128)])
            return 0
        lax.fori_loop(0, NN // NS // 128, dump, 0)
        plsc.subcore_barrier()


def _agg_kernel(d):
    nvb = d // 128
    mesh = plsc.VectorSubcoreMesh(core_axis_name="c", subcore_axis_name="s")
    return pl.kernel(
        functools.partial(_agg_body, nvb),
        out_type=tuple(
            jax.ShapeDtypeStruct((2, NN, 128), jnp.float32)
            for _ in range(nvb)),
        mesh=mesh,
        scratch_types=[
            pltpu.VMEM((2 * CHC,), jnp.int32),
            pltpu.VMEM((2 * CHC,), jnp.int32),
            pltpu.VMEM((CHC,), jnp.int32),
            pltpu.VMEM((CHC,), jnp.int32),
            pltpu.VMEM((CHC,), jnp.int32),
            pltpu.VMEM((CHC,), jnp.int32),
            pltpu.VMEM((CHC,), jnp.float32),
            pltpu.VMEM((CHC,), jnp.float32),
            pltpu.VMEM((CHC, 128), jnp.float32),
            pltpu.VMEM((CHC, 128), jnp.float32),
            pltpu.VMEM((CHC, 128), jnp.float32),
            pltpu.VMEM((CHC, 128), jnp.float32),
            pltpu.VMEM((8, 128), jnp.float32),
            pltpu.VMEM_SHARED((NN, 128), jnp.float32),
            pltpu.SemaphoreType.DMA,
            pltpu.SemaphoreType.DMA,
            pltpu.SemaphoreType.DMA,
            pltpu.SemaphoreType.DMA,
            pltpu.SemaphoreType.DMA,
            pltpu.SemaphoreType.DMA,
        ],
        compiler_params=_SC_PARAMS,
    )


def _layer(xs, p, comb, dst_p, act):
    d = p["Wq"].shape[1]
    w = jnp.concatenate([p["Wq"], p["Wk"], p["Wv"], p["Ws"]], axis=1)
    b = jnp.concatenate([p["bq"], p["bk"], p["bv"], p["bs"]])
    outs = _qkvs_matmul(xs, w, b, act)
    qk, skip = outs[0], outs[1]
    v_blocks = outs[2:]
    alpha, mpart, spart = _alpha_kernel(d)(qk.reshape(2 * NN, d), comb[0])
    mg, sg = _merge_kernel()(mpart.reshape(-1), spart.reshape(-1))
    wgt = _wgt_kernel()(dst_p, alpha, mg, sg)
    agg_parts = _agg_kernel(d)(*v_blocks, comb[1], wgt)
    if not isinstance(agg_parts, (list, tuple)):
        agg_parts = (agg_parts,)
    return _combine(skip, agg_parts)


def kernel(features, img_feat, edge_index, batch_index, params):
    pad_n = NN - N_NODES
    feat_p = jnp.pad(features, ((0, pad_n), (0, 0)))
    img_p = jnp.pad(img_feat, ((0, pad_n), (0, 0)))
    src_p = jnp.pad(edge_index[0], (0, EPAD - N_EDGES))
    dst_p = jnp.pad(edge_index[1], (0, EPAD - N_EDGES),
                    constant_values=N_NODES)
    batch_p = jnp.pad(batch_index, (0, pad_n), constant_values=N_GRAPHS)
    comb_a = jnp.concatenate(
        [dst_p.reshape(-1, CHA), src_p.reshape(-1, CHA) + NN],
        axis=1).reshape(-1)
    comb_c = jnp.concatenate(
        [dst_p.reshape(-1, CHC), src_p.reshape(-1, CHC) + NN],
        axis=1).reshape(-1)
    comb = (comb_a, comb_c)

    h1 = _layer((feat_p,), params["conv1"], comb, dst_p, act=False)
    h2 = _layer((h1,), params["conv2"], comb, dst_p, act=True)
    h3p = _layer((h2,), params["conv3"], comb, dst_p, act=False)
    h4 = _layer((h3p,), params["conv4"], comb, dst_p, act=True)
    img1p = _layer((img_p,), params["imgconv1"], comb, dst_p, act=False)
    img2 = _layer((img1p,), params["imgconv2"], comb, dst_p, act=True)
    img3p = _layer((img2,), params["imgconv3"], comb, dst_p, act=False)
    img4 = _layer((img3p,), params["imgconv4"], comb, dst_p, act=True)
    combine_p = _layer((h2, img2), params["neck"], comb, dst_p, act=False)
    c2 = _layer((combine_p,), params["neck2"], comb, dst_p, act=True)
    c3p = _layer((c2,), params["c3"], comb, dst_p, act=False)
    c4 = _layer((c3p,), params["c4"], comb, dst_p, act=True)
    hidden = _gap(c2, batch_p)
    return (h2[:N_NODES], img2[:N_NODES], c2[:N_NODES], h4[:N_NODES],
            img4[:N_NODES], c4[:N_NODES], hidden)


# speedup vs baseline: 2.0768x; 1.0728x over previous
import functools

import jax
import jax.numpy as jnp
from jax import lax
from jax.experimental import pallas as pl
from jax.experimental.pallas import tpu as pltpu
from jax.experimental.pallas import tpu_sc as plsc

N_NODES = 10000
N_EDGES = 160000
N_GRAPHS = 16

NN = 10240
EPAD = 163840
NW = 32
NS = 16
EP = EPAD // NW
CH = 128
NCH = EP // CH
NP = NN // NW
NEG = -3.0e38

_ROW_BLK = 640

_SC_PARAMS = pltpu.CompilerParams(needs_layout_passes=False)


def _qkvs_body(nx, act, d, nvb, *refs):
    x_refs = refs[:nx]
    w_ref, b_ref = refs[nx], refs[nx + 1]
    outs = refs[nx + 2:]
    x = x_refs[0][...] if nx == 1 else jnp.concatenate(
        [r[...] for r in x_refs], axis=1)
    if act:
        x = jnp.where(x > 0, x, jnp.exp(jnp.minimum(x, 0.0)) - 1.0)
    full = jnp.dot(x, w_ref[...], preferred_element_type=jnp.float32) + b_ref[...]
    outs[0][0] = full[:, :d]
    outs[0][1] = full[:, d:2 * d]
    outs[1][...] = full[:, 3 * d:]
    for i in range(nvb):
        outs[2 + i][...] = full[:, 2 * d + i * 128:2 * d + (i + 1) * 128]


def _qkvs_matmul(xs, w, b, act):
    nx = len(xs)
    din = sum(x.shape[1] for x in xs)
    d = w.shape[1] // 4
    nvb = d // 128
    grid = NN // _ROW_BLK
    in_specs = [pl.BlockSpec((_ROW_BLK, x.shape[1]), lambda i: (i, 0)) for x in xs]
    in_specs += [
        pl.BlockSpec((din, 4 * d), lambda i: (0, 0)),
        pl.BlockSpec((1, 4 * d), lambda i: (0, 0)),
    ]
    out_specs = [pl.BlockSpec((2, _ROW_BLK, d), lambda i: (0, i, 0)),
                 pl.BlockSpec((_ROW_BLK, d), lambda i: (i, 0))] + [
        pl.BlockSpec((_ROW_BLK, 128), lambda i: (i, 0))] * nvb
    out_shape = [jax.ShapeDtypeStruct((2, NN, d), jnp.float32),
                 jax.ShapeDtypeStruct((NN, d), jnp.float32)] + [
        jax.ShapeDtypeStruct((NN, 128), jnp.float32)] * nvb
    return pl.pallas_call(
        functools.partial(_qkvs_body, nx, act, d, nvb),
        grid=(grid,),
        in_specs=in_specs,
        out_specs=out_specs,
        out_shape=out_shape,
    )(*xs, w, b.reshape(1, -1))


def _combine_body(nvb, *refs):
    s_ref = refs[0]
    aggs = refs[1:1 + nvb]
    o_ref = refs[1 + nvb]
    out = s_ref[...]
    parts = []
    for i in range(nvb):
        parts.append(aggs[i][0] + aggs[i][1])
    o_ref[...] = out + jnp.concatenate(parts, axis=1)


def _combine(skip, agg_parts):
    d = skip.shape[1]
    nvb = d // 128
    grid = NN // _ROW_BLK
    in_specs = [pl.BlockSpec((_ROW_BLK, d), lambda i: (i, 0))] + [
        pl.BlockSpec((2, _ROW_BLK, 128), lambda i: (0, i, 0))] * nvb
    return pl.pallas_call(
        functools.partial(_combine_body, nvb),
        grid=(grid,),
        in_specs=in_specs,
        out_specs=pl.BlockSpec((_ROW_BLK, d), lambda i: (i, 0)),
        out_shape=jax.ShapeDtypeStruct((NN, d), jnp.float32),
    )(skip, *agg_parts)


def _gap_body(c2_ref, b_ref, o_ref):
    gids = lax.broadcasted_iota(jnp.int32, (N_GRAPHS, NN), 0)
    mask = (b_ref[...] == gids).astype(jnp.float32)
    sums = jnp.dot(mask, c2_ref[...], preferred_element_type=jnp.float32)
    counts = jnp.sum(mask, axis=1, keepdims=True)
    o_ref[...] = sums / jnp.maximum(counts, 1.0)


def _gap(c2_pad, batch_pad):
    return pl.pallas_call(
        _gap_body,
        in_specs=[
            pl.BlockSpec((NN, c2_pad.shape[1]), lambda: (0, 0)),
            pl.BlockSpec((1, NN), lambda: (0, 0)),
        ],
        out_specs=pl.BlockSpec((N_GRAPHS, c2_pad.shape[1]), lambda: (0, 0)),
        out_shape=jax.ShapeDtypeStruct((N_GRAPHS, c2_pad.shape[1]), jnp.float32),
    )(c2_pad, batch_pad.reshape(1, -1))


_GDN = lax.GatherDimensionNumbers(
    offset_dims=(), collapsed_slice_dims=(0,), start_index_map=(0,))


def _lane_shift(x, s):
    idx = (lax.iota(jnp.int32, 16) + s) & 15
    return lax.gather(x, idx[:, None], dimension_numbers=_GDN,
                      slice_sizes=(1,),
                      mode=lax.GatherScatterMode.PROMISE_IN_BOUNDS)


def _exp0(x):
    return jnp.exp(jnp.maximum(x, -87.0))


def _butterfly_softmax(key, m, s):
    for sh in (1, 2, 4, 8):
        k2 = _lane_shift(key, sh)
        m2 = jnp.where(key == k2, _lane_shift(m, sh), NEG)
        s2 = jnp.where(key == k2, _lane_shift(s, sh), 0.0)
        mm = jnp.maximum(m, m2)
        s = s * _exp0(m - mm) + s2 * _exp0(m2 - mm)
        m = mm
    return m, s


def _worker_id():
    return lax.axis_index("c") * NS + lax.axis_index("s")


CHA = 32
NCHA = EP // CHA
NBA = 4


def _alpha_body(d, qk_hbm, comb_hbm, alpha_hbm, mpart_hbm, spart_hbm,
                idxv0, idxv1, idxv2, idxv3, dstc0, dstc1, dstc2, dstc3,
                qkbuf0, qkbuf1, qkbuf2, qkbuf3,
                abuf0, abuf1, abuf2, abuf3, mloc, sloc,
                semi0, semi1, semi2, semi3,
                semg0, semg1, semg2, semg3,
                sema0, sema1, sema2, sema3):
    w = _worker_id()
    scale = 1.0 / float(d) ** 0.5
    idxv = (idxv0, idxv1, idxv2, idxv3)
    dstc = (dstc0, dstc1, dstc2, dstc3)
    qkbuf = (qkbuf0, qkbuf1, qkbuf2, qkbuf3)
    abuf = (abuf0, abuf1, abuf2, abuf3)
    semi = (semi0, semi1, semi2, semi3)
    semg = (semg0, semg1, semg2, semg3)
    sema = (sema0, sema1, sema2, sema3)

    def init(i, _):
        mloc[pl.ds(i * 16, 16)] = jnp.full((16,), NEG, jnp.float32)
        sloc[pl.ds(i * 16, 16)] = jnp.zeros((16,), jnp.float32)
        return 0
    lax.fori_loop(0, NN // 16, init, 0)

    def issue_idx(ci, b):
        base = (w * NCHA + ci) * 2 * CHA
        pltpu.async_copy(comb_hbm.at[pl.ds(base, 2 * CHA)], idxv[b], semi[b])

    def wait_idx(b):
        pltpu.make_async_copy(comb_hbm.at[pl.ds(0, 2 * CHA)], idxv[b],
                              semi[b]).wait()

    def issue_gather(b):
        pltpu.async_copy(qk_hbm.at[idxv[b]], qkbuf[b], semg[b])

    def wait_gather(b):
        pltpu.make_async_copy(qk_hbm.at[idxv[b]], qkbuf[b], semg[b]).wait()

    def wait_alpha(b):
        pltpu.make_async_copy(abuf[b], alpha_hbm.at[pl.ds(0, CHA)],
                              sema[b]).wait()

    def compute(ci, b):
        def grp(g, _):
            iota = lax.iota(jnp.int32, 16)
            alpha = jnp.zeros((16,), jnp.float32)
            for u in range(16):
                r = g * 16 + u

                def rowdot(jo, acc, _r=r):
                    for jj in range(8):
                        sl = pl.ds(jo * 128 + jj * 16, 16)
                        acc = acc + qkbuf[b][_r, sl] * qkbuf[b][_r + CHA, sl]
                    return acc
                acc = lax.fori_loop(0, d // 128, rowdot,
                                    jnp.zeros((16,), jnp.float32))
                for sh in (1, 2, 4, 8):
                    acc = acc + _lane_shift(acc, sh)
                alpha = jnp.where(iota == u, acc, alpha)
            alpha = alpha * scale
            dst16 = dstc[b][pl.ds(g * 16, 16)]
            m, ss = _butterfly_softmax(dst16, alpha,
                                       jnp.ones((16,), jnp.float32))
            curm = plsc.load_gather(mloc, [dst16])
            curs = plsc.load_gather(sloc, [dst16])
            mm = jnp.maximum(curm, m)
            snew = curs * _exp0(curm - mm) + ss * _exp0(m - mm)
            plsc.store_scatter(mloc, [dst16], mm)
            plsc.store_scatter(sloc, [dst16], snew)
            abuf[b][pl.ds(g * 16, 16)] = alpha
            return 0
        lax.fori_loop(0, CHA // 16, grp, 0)
        pltpu.async_copy(abuf[b], alpha_hbm.at[pl.ds(w * EP + ci * CHA, CHA)],
                         sema[b])

    for b in range(NBA):
        issue_idx(b, b)
    for b in range(NBA):
        wait_idx(b)
        issue_gather(b)
    nloops = NCHA // NBA

    def ring(i, _):
        c0 = NBA * i
        for b in range(NBA):
            c = c0 + b
            wait_gather(b)
            for h in range(CHA // 16):
                hs = pl.ds(h * 16, 16)
                dstc[b][hs] = idxv[b][hs]

            @pl.when(i < nloops - 1)
            def _(b=b, c=c):
                issue_idx(c + NBA, b)

            @pl.when(i > 0)
            def _(b=b):
                wait_alpha(b)
            compute(c, b)

            @pl.when(i < nloops - 1)
            def _(b=b):
                wait_idx(b)
                issue_gather(b)
        return 0
    lax.fori_loop(0, nloops, ring, 0)
    for b in range(NBA):
        wait_alpha(b)
    pltpu.sync_copy(mloc, mpart_hbm.at[w])
    pltpu.sync_copy(sloc, spart_hbm.at[w])


def _alpha_kernel(d):
    mesh = plsc.VectorSubcoreMesh(core_axis_name="c", subcore_axis_name="s")
    return pl.kernel(
        functools.partial(_alpha_body, d),
        out_type=(
            jax.ShapeDtypeStruct((EPAD,), jnp.float32),
            jax.ShapeDtypeStruct((NW, NN), jnp.float32),
            jax.ShapeDtypeStruct((NW, NN), jnp.float32),
        ),
        mesh=mesh,
        scratch_types=(
            [pltpu.VMEM((2 * CHA,), jnp.int32)] * 4
            + [pltpu.VMEM((CHA,), jnp.int32)] * 4
            + [pltpu.VMEM((2 * CHA, d), jnp.float32)] * 4
            + [pltpu.VMEM((CHA,), jnp.float32)] * 4
            + [pltpu.VMEM((NN,), jnp.float32)] * 2
            + [pltpu.SemaphoreType.DMA] * 12
        ),
        compiler_params=_SC_PARAMS,
    )


CHW = 1024
NCHW = EP // CHW


def _wgt_body(dst_hbm, alpha_hbm, mg_hbm, sg_hbm, wgt_hbm,
              dstb, ab, wb, mv, sv):
    w = _worker_id()
    pltpu.sync_copy(mg_hbm, mv)
    pltpu.sync_copy(sg_hbm, sv)

    def chunk(ci, _):
        base = w * EP + ci * CHW
        pltpu.sync_copy(dst_hbm.at[pl.ds(base, CHW)], dstb)
        pltpu.sync_copy(alpha_hbm.at[pl.ds(base, CHW)], ab)

        def grp(g, _):
            sl = pl.ds(g * 16, 16)
            dst16 = dstb[sl]
            m16 = plsc.load_gather(mv, [dst16])
            s16 = plsc.load_gather(sv, [dst16])
            wb[sl] = _exp0(ab[sl] - m16) / (s16 + 1e-16)
            return 0
        lax.fori_loop(0, CHW // 16, grp, 0)
        pltpu.sync_copy(wb, wgt_hbm.at[pl.ds(base, CHW)])
        return 0
    lax.fori_loop(0, NCHW, chunk, 0)


def _wgt_kernel():
    mesh = plsc.VectorSubcoreMesh(core_axis_name="c", subcore_axis_name="s")
    return pl.kernel(
        _wgt_body,
        out_type=jax.ShapeDtypeStruct((EPAD,), jnp.float32),
        mesh=mesh,
        scratch_types=[
            pltpu.VMEM((CHW,), jnp.int32),
            pltpu.VMEM((CHW,), jnp.float32),
            pltpu.VMEM((CHW,), jnp.float32),
            pltpu.VMEM((NN,), jnp.float32),
            pltpu.VMEM((NN,), jnp.float32),
        ],
        compiler_params=_SC_PARAMS,
    )


def _merge_body(mpart_hbm, spart_hbm, mg_hbm, sg_hbm, blkm, blks, mgv, sgv):
    w = _worker_id()
    for t in range(NW):
        pltpu.sync_copy(mpart_hbm.at[pl.ds(t * NN + w * NP, NP)],
                        blkm.at[pl.ds(t * NP, NP)])
        pltpu.sync_copy(spart_hbm.at[pl.ds(t * NN + w * NP, NP)],
                        blks.at[pl.ds(t * NP, NP)])

    def col(i, _):
        m = jnp.full((16,), NEG, jnp.float32)
        for t in range(NW):
            m = jnp.maximum(m, blkm[pl.ds(t * NP + i * 16, 16)])
        s = jnp.zeros((16,), jnp.float32)
        for t in range(NW):
            mt = blkm[pl.ds(t * NP + i * 16, 16)]
            s = s + blks[pl.ds(t * NP + i * 16, 16)] * _exp0(mt - m)
        mgv[pl.ds(i * 16, 16)] = m
        sgv[pl.ds(i * 16, 16)] = s
        return 0
    lax.fori_loop(0, NP // 16, col, 0)
    pltpu.sync_copy(mgv, mg_hbm.at[pl.ds(w * NP, NP)])
    pltpu.sync_copy(sgv, sg_hbm.at[pl.ds(w * NP, NP)])


def _merge_kernel():
    mesh = plsc.VectorSubcoreMesh(core_axis_name="c", subcore_axis_name="s")
    return pl.kernel(
        _merge_body,
        out_type=(
            jax.ShapeDtypeStruct((NN,), jnp.float32),
            jax.ShapeDtypeStruct((NN,), jnp.float32),
        ),
        mesh=mesh,
        scratch_types=[
            pltpu.VMEM((NW * NP,), jnp.float32),
            pltpu.VMEM((NW * NP,), jnp.float32),
            pltpu.VMEM((NP,), jnp.float32),
            pltpu.VMEM((NP,), jnp.float32),
        ],
        compiler_params=_SC_PARAMS,
    )


CHC = 32
NCHC = EP // CHC
NBC = 4


def _agg_body(nvb, *refs):
    v_blocks = refs[:nvb]
    comb_hbm, wgt_hbm = refs[nvb:nvb + 2]
    agg_outs = refs[nvb + 2:nvb + 2 + nvb]
    sc = refs[nvb + 2 + nvb:]
    idxv, srcv, dsts, wb = sc[0:4], sc[4:8], sc[8:12], sc[12:16]
    vbuf, sbuf = sc[16:20], sc[20:24]
    zbuf, aggsp = sc[24], sc[25]
    semi, semg, sems = sc[26:30], sc[30:34], sc[34:38]
    cid = lax.axis_index("c")
    sid = lax.axis_index("s")
    w = cid * NS + sid

    def zrow(r, _):
        for jj in range(8):
            zbuf[r, pl.ds(jj * 16, 16)] = jnp.zeros((16,), jnp.float32)
        return 0
    lax.fori_loop(0, 8, zrow, 0)

    def issue_idx(ci, b):
        pltpu.async_copy(comb_hbm.at[pl.ds((w * NCHC + ci) * 2 * CHC,
                                           2 * CHC)], idxv[b], semi[b])
        pltpu.async_copy(wgt_hbm.at[pl.ds(w * EP + ci * CHC, CHC)],
                         wb[b], semi[b])

    def wait_idx_derive_src(b):
        pltpu.make_async_copy(comb_hbm.at[pl.ds(0, 2 * CHC)], idxv[b],
                              semi[b]).wait()
        pltpu.make_async_copy(wgt_hbm.at[pl.ds(0, CHC)], wb[b],
                              semi[b]).wait()
        for h in range(CHC // 16):
            hs = pl.ds(h * 16, 16)
            srcv[b][hs] = idxv[b][pl.ds(CHC + h * 16, 16)] - NN

    def wait_scat(b):
        pltpu.make_async_copy(sbuf[b], aggsp.at[dsts[b]], sems[b]).wait()

    for blk in range(nvb):
        vb_hbm = v_blocks[blk]

        def issue_gather(b, _vb=vb_hbm):
            pltpu.async_copy(_vb.at[srcv[b]], vbuf[b], semg[b])

        def wait_gather(b, _vb=vb_hbm):
            pltpu.make_async_copy(_vb.at[srcv[b]], vbuf[b], semg[b]).wait()

        def process(ci, b):
            def grp(g, _):
                w16 = wb[b][pl.ds(g * 16, 16)]
                for u in range(16):
                    r = g * 16 + u
                    wv = jnp.full((16,), w16[u])
                    for jj in range(8):
                        cs = pl.ds(jj * 16, 16)
                        sbuf[b][r, cs] = vbuf[b][r, cs] * wv
                return 0
            lax.fori_loop(0, CHC // 16, grp, 0)
            for h in range(CHC // 16):
                hs = pl.ds(h * 16, 16)
                dsts[b][hs] = idxv[b][hs]
            pltpu.async_copy(sbuf[b], aggsp.at[dsts[b]], sems[b], add=True)

        def zsp(i, _):
            pltpu.sync_copy(zbuf, aggsp.at[pl.ds(sid * (NN // NS) + i * 8, 8)])
            return 0
        lax.fori_loop(0, NN // NS // 8, zsp, 0)
        plsc.subcore_barrier()

        for b in range(NBC):
            issue_idx(b, b)
        for b in range(NBC):
            wait_idx_derive_src(b)
            issue_gather(b)
        nloops = NCHC // NBC

        def ring(i, _):
            c0 = NBC * i
            for b in range(NBC):
                c = c0 + b
                wait_gather(b)

                @pl.when(i > 0)
                def _(b=b):
                    wait_scat(b)
                process(c, b)

                @pl.when(i < nloops - 1)
                def _(b=b, c=c):
                    issue_idx(c + NBC, b)
                    wait_idx_derive_src(b)
                    issue_gather(b)
            return 0
        lax.fori_loop(0, nloops, ring, 0)
        for b in range(NBC):
            wait_scat(b)
        plsc.subcore_barrier()

        def dump(i, _):
            rowbase = sid * (NN // NS) + i * 128
            pltpu.sync_copy(aggsp.at[pl.ds(rowbase, 128)],
                            agg_outs[blk].at[cid, pl.ds(rowbase, 128)])
            return 0
        lax.fori_loop(0, NN // NS // 128, dump, 0)
        plsc.subcore_barrier()


def _agg_kernel(d):
    nvb = d // 128
    mesh = plsc.VectorSubcoreMesh(core_axis_name="c", subcore_axis_name="s")
    return pl.kernel(
        functools.partial(_agg_body, nvb),
        out_type=tuple(
            jax.ShapeDtypeStruct((2, NN, 128), jnp.float32)
            for _ in range(nvb)),
        mesh=mesh,
        scratch_types=(
            [pltpu.VMEM((2 * CHC,), jnp.int32)] * 4
            + [pltpu.VMEM((CHC,), jnp.int32)] * 8
            + [pltpu.VMEM((CHC,), jnp.float32)] * 4
            + [pltpu.VMEM((CHC, 128), jnp.float32)] * 8
            + [pltpu.VMEM((8, 128), jnp.float32)]
            + [pltpu.VMEM_SHARED((NN, 128), jnp.float32)]
            + [pltpu.SemaphoreType.DMA] * 12
        ),
        compiler_params=_SC_PARAMS,
    )


def _layer(xs, p, comb, dst_p, act):
    d = p["Wq"].shape[1]
    w = jnp.concatenate([p["Wq"], p["Wk"], p["Wv"], p["Ws"]], axis=1)
    b = jnp.concatenate([p["bq"], p["bk"], p["bv"], p["bs"]])
    outs = _qkvs_matmul(xs, w, b, act)
    qk, skip = outs[0], outs[1]
    v_blocks = outs[2:]
    alpha, mpart, spart = _alpha_kernel(d)(qk.reshape(2 * NN, d), comb[0])
    mg, sg = _merge_kernel()(mpart.reshape(-1), spart.reshape(-1))
    wgt = _wgt_kernel()(dst_p, alpha, mg, sg)
    agg_parts = _agg_kernel(d)(*v_blocks, comb[1], wgt)
    if not isinstance(agg_parts, (list, tuple)):
        agg_parts = (agg_parts,)
    return _combine(skip, agg_parts)


def kernel(features, img_feat, edge_index, batch_index, params):
    pad_n = NN - N_NODES
    feat_p = jnp.pad(features, ((0, pad_n), (0, 0)))
    img_p = jnp.pad(img_feat, ((0, pad_n), (0, 0)))
    src_p = jnp.pad(edge_index[0], (0, EPAD - N_EDGES))
    dst_p = jnp.pad(edge_index[1], (0, EPAD - N_EDGES),
                    constant_values=N_NODES)
    batch_p = jnp.pad(batch_index, (0, pad_n), constant_values=N_GRAPHS)
    comb_a = jnp.concatenate(
        [dst_p.reshape(-1, CHA), src_p.reshape(-1, CHA) + NN],
        axis=1).reshape(-1)
    comb_c = jnp.concatenate(
        [dst_p.reshape(-1, CHC), src_p.reshape(-1, CHC) + NN],
        axis=1).reshape(-1)
    comb = (comb_a, comb_c)

    h1 = _layer((feat_p,), params["conv1"], comb, dst_p, act=False)
    h2 = _layer((h1,), params["conv2"], comb, dst_p, act=True)
    h3p = _layer((h2,), params["conv3"], comb, dst_p, act=False)
    h4 = _layer((h3p,), params["conv4"], comb, dst_p, act=True)
    img1p = _layer((img_p,), params["imgconv1"], comb, dst_p, act=False)
    img2 = _layer((img1p,), params["imgconv2"], comb, dst_p, act=True)
    img3p = _layer((img2,), params["imgconv3"], comb, dst_p, act=False)
    img4 = _layer((img3p,), params["imgconv4"], comb, dst_p, act=True)
    combine_p = _layer((h2, img2), params["neck"], comb, dst_p, act=False)
    c2 = _layer((combine_p,), params["neck2"], comb, dst_p, act=True)
    c3p = _layer((c2,), params["c3"], comb, dst_p, act=False)
    c4 = _layer((c3p,), params["c4"], comb, dst_p, act=True)
    hidden = _gap(c2, batch_p)
    return (h2[:N_NODES], img2[:N_NODES], c2[:N_NODES], h4[:N_NODES],
            img4[:N_NODES], c4[:N_NODES], hidden)
